# baseline (speedup 1.0000x reference)
import functools

import jax
import jax.numpy as jnp
from jax import lax
from jax.experimental import pallas as pl
from jax.experimental.pallas import tpu as pltpu
from jax.experimental.pallas import tpu_sc as plsc

B, K, L = 4096, 64, 200
KB = 2

NC, NS = 2, 16
NW = NC * NS
COLS = B // NW
VECS = COLS // 16


def _tc_body(x_ref, o_ref):
    xi = x_ref[...].astype(jnp.int32)
    kbase = pl.program_id(0) * KB
    k = kbase + jax.lax.broadcasted_iota(jnp.int32, (KB, L, B), 0)
    o_ref[...] = jnp.where(k == xi, 0.0, -jnp.inf)


def _kernel_tc(xt):
    return pl.pallas_call(
        _tc_body,
        grid=(K // KB,),
        in_specs=[pl.BlockSpec((1, L, B), lambda i: (0, 0, 0))],
        out_specs=pl.BlockSpec((KB, L, B), lambda i: (i, 0, 0)),
        out_shape=jax.ShapeDtypeStruct((K, L, B), jnp.float32),
    )(xt)


@functools.partial(
    pl.kernel,
    mesh=plsc.VectorSubcoreMesh(core_axis_name="c", subcore_axis_name="s"),
    out_type=jax.ShapeDtypeStruct((K, L, B), jnp.float32),
    scratch_types=[
        pltpu.VMEM((L, COLS), jnp.float32),
        pltpu.VMEM((L, COLS), jnp.float32),
        pltpu.VMEM((L, COLS), jnp.float32),
        pltpu.SemaphoreType.DMA,
        pltpu.SemaphoreType.DMA,
    ],
)
def _kernel_sc(x_hbm, out_hbm, x_v, ob0, ob1, sem0, sem1):
    wid = lax.axis_index("s") * NC + lax.axis_index("c")
    col0 = wid * COLS
    pltpu.sync_copy(x_hbm.at[0, :, pl.ds(col0, COLS)], x_v)
    bufs, sems = (ob0, ob1), (sem0, sem1)
    pending = [None, None]
    for k in range(K):
        buf, sem = bufs[k % 2], sems[k % 2]
        if pending[k % 2] is not None:
            pending[k % 2].wait()
        kf = jnp.float32(k)

        def _row(r, _, buf=buf):
            for c in range(VECS):
                xv = x_v[r, pl.ds(c * 16, 16)]
                buf[r, pl.ds(c * 16, 16)] = jnp.where(xv == kf, 0.0, -jnp.inf)
            return 0

        lax.fori_loop(0, L, _row, 0)
        pending[k % 2] = pltpu.async_copy(
            buf, out_hbm.at[k, :, pl.ds(col0, COLS)], sem
        )
    pending[0].wait()
    pending[1].wait()


def kernel(x):
    xt = jnp.transpose(x, (1, 2, 0))
    out = _kernel_sc(xt)
    return jnp.transpose(out, (2, 0, 1))


# speedup vs baseline: 1.3709x; 1.3709x over previous
import functools

import jax
import jax.numpy as jnp
from jax import lax
from jax.experimental import pallas as pl
from jax.experimental.pallas import tpu as pltpu
from jax.experimental.pallas import tpu_sc as plsc

B, K, L = 4096, 64, 200
KB = 2

NC, NS = 2, 16
NW = NC * NS
COLS = B // NW
VECS = COLS // 16


def _tc_body(x_ref, o_ref):
    xi = x_ref[...].astype(jnp.int32)
    kbase = pl.program_id(0) * KB
    k = kbase + jax.lax.broadcasted_iota(jnp.int32, (KB, L, B), 0)
    o_ref[...] = jnp.where(k == xi, 0.0, -jnp.inf)


def _kernel_tc(xt):
    return pl.pallas_call(
        _tc_body,
        grid=(K // KB,),
        in_specs=[pl.BlockSpec((1, L, B), lambda i: (0, 0, 0))],
        out_specs=pl.BlockSpec((KB, L, B), lambda i: (i, 0, 0)),
        out_shape=jax.ShapeDtypeStruct((K, L, B), jnp.float32),
    )(xt)


@functools.partial(
    pl.kernel,
    mesh=plsc.VectorSubcoreMesh(core_axis_name="c", subcore_axis_name="s"),
    out_type=jax.ShapeDtypeStruct((K, L, B), jnp.float32),
    scratch_types=[
        pltpu.VMEM((L, COLS), jnp.float32),
        pltpu.VMEM((L, COLS), jnp.float32),
        pltpu.VMEM((L, COLS), jnp.float32),
        pltpu.SemaphoreType.DMA,
        pltpu.SemaphoreType.DMA,
    ],
)
def _kernel_sc(x_hbm, out_hbm, x_v, ob0, ob1, sem0, sem1):
    wid = lax.axis_index("s") * NC + lax.axis_index("c")
    col0 = wid * COLS
    pltpu.sync_copy(x_hbm.at[0, :, pl.ds(col0, COLS)], x_v)
    bufs, sems = (ob0, ob1), (sem0, sem1)
    pending = [None, None]
    for k in range(K):
        buf, sem = bufs[k % 2], sems[k % 2]
        if pending[k % 2] is not None:
            pending[k % 2].wait()
        kf = jnp.float32(k)

        def _row(r, _, buf=buf):
            for c in range(VECS):
                xv = x_v[r, pl.ds(c * 16, 16)]
                buf[r, pl.ds(c * 16, 16)] = jnp.where(xv == kf, 0.0, -jnp.inf)
            return 0

        lax.fori_loop(0, L, _row, 0)
        pending[k % 2] = pltpu.async_copy(
            buf, out_hbm.at[k, :, pl.ds(col0, COLS)], sem
        )
    pending[0].wait()
    pending[1].wait()


NBUF = 4


def _mdma_body(x_hbm, o_hbm, xv, b0, b1, b2, b3, sin, s0, s1, s2, s3):
    pltpu.make_async_copy(x_hbm, xv, sin).start()
    pltpu.make_async_copy(x_hbm, xv, sin).wait()
    bufs, sems = (b0, b1, b2, b3), (s0, s1, s2, s3)

    def _plane(i, _):
        for j in range(NBUF):
            k = i * NBUF + j

            @pl.when(i > 0)
            def _():
                pltpu.make_async_copy(bufs[j], o_hbm.at[k - NBUF], sems[j]).wait()

            kf = k.astype(jnp.float32) if hasattr(k, "astype") else float(k)
            bufs[j][...] = jnp.where(xv[...] == kf, 0.0, -jnp.inf)
            pltpu.make_async_copy(bufs[j], o_hbm.at[k], sems[j]).start()
        return 0

    lax.fori_loop(0, K // NBUF, _plane, 0)
    for j in range(NBUF):
        pltpu.make_async_copy(bufs[j], o_hbm.at[K - NBUF + j], sems[j]).wait()


def _kernel_mdma(xt2d):
    return pl.pallas_call(
        _mdma_body,
        in_specs=[pl.BlockSpec(memory_space=pl.ANY)],
        out_specs=pl.BlockSpec(memory_space=pl.ANY),
        out_shape=jax.ShapeDtypeStruct((K, L, B), jnp.float32),
        scratch_shapes=(
            [pltpu.VMEM((L, B), jnp.float32)] * (1 + NBUF)
            + [pltpu.SemaphoreType.DMA] * (1 + NBUF)
        ),
    )(xt2d)


def kernel(x):
    xt = jnp.transpose(x, (1, 2, 0))
    out = _kernel_mdma(jnp.reshape(xt, (L, B)))
    return jnp.transpose(out, (2, 0, 1))


# speedup vs baseline: 1.3711x; 1.0001x over previous
import jax
import jax.numpy as jnp
from jax.experimental import pallas as pl

B, K, L = 4096, 64, 200
KB = 2


def _body(x_ref, o_ref):
    xi = x_ref[...].astype(jnp.int32)
    kbase = pl.program_id(0) * KB
    k = kbase + jax.lax.broadcasted_iota(jnp.int32, (KB, L, B), 0)
    o_ref[...] = jnp.where(k == xi, 0.0, -jnp.inf)


def kernel(x):
    xt = jnp.transpose(x, (1, 2, 0))
    out = pl.pallas_call(
        _body,
        grid=(K // KB,),
        in_specs=[pl.BlockSpec((1, L, B), lambda i: (0, 0, 0))],
        out_specs=pl.BlockSpec((KB, L, B), lambda i: (i, 0, 0)),
        out_shape=jax.ShapeDtypeStruct((K, L, B), jnp.float32),
    )(xt)
    return jnp.transpose(out, (2, 0, 1))
